# baseline (speedup 1.0000x reference)
import jax
import jax.numpy as jnp
from jax import lax
from jax.experimental import pallas as pl
from jax.experimental.pallas import tpu as pltpu
from jax.experimental.pallas import tpu_sc as plsc

N_NODES = 10000
HALF = 128
N_EDGES = 160000
N_BATCH = 16384
N_LAYERS = 3

NC = 2
NS = 16
L = 16

NP = 10240
CHUNK = NP // NS
RB = 32
NRB = CHUNK // RB
K = 64
SB = 16
EPW = 10240
NB = EPW // K
NSB = NB // SB
NGB = 4
EPAD = EPW * NS
EREAL = N_EDGES // NS
NPADE = EPW - EREAL

PPW = N_BATCH // (NC * NS)
PB = PPW // K


def _rsqrt16(v):
    iv = plsc.bitcast(v, jnp.int32)
    iv = jnp.int32(0x5F3759DF) - (iv >> 1)
    y = plsc.bitcast(iv, jnp.float32)
    for _ in range(3):
        y = y * (1.5 - 0.5 * v * y * y)
    return jnp.where(v > 0.0, y, 0.0)


def _prop_body(srcp, dstp, emb2, acc_out, y_hbm,
               accum, deg, sidx, didx, g0, g1, g2, g3, onesb, disv,
               abuf, obuf, sg0, sg1, sg2, sg3, ss0, ss1, ss2, ss3):
    gbufs = (g0, g1, g2, g3)
    gsems = (sg0, sg1, sg2, sg3)
    ssems = (ss0, ss1, ss2, ss3)
    c = lax.axis_index("c")
    s = lax.axis_index("s")
    base = s * CHUNK

    one = jnp.ones((L,), jnp.float32)
    zv = jnp.zeros((L,), jnp.float32)
    for t in range(K // L):
        onesb[pl.ds(t * L, L)] = one

    @pl.loop(0, K)
    def _(r):
        for t in range(HALF // L):
            g0[r, pl.ds(t * L, L)] = zv

    @pl.loop(0, CHUNK // L)
    def _(i):
        disv[pl.ds(i * L, L)] = zv

    pltpu.sync_copy(disv, deg.at[pl.ds(base, CHUNK)])

    for i in range(CHUNK // K):
        pltpu.sync_copy(g0, accum.at[pl.ds(base + i * K, K)])

    plsc.subcore_barrier()

    with jax.named_scope("deg_phase"):
        @pl.loop(0, NSB)
        def _(q):
            pltpu.sync_copy(dstp.at[s].at[q], didx)
            for b in range(SB):
                pltpu.sync_copy(onesb, deg.at[didx.at[b]], add=True)

    plsc.subcore_barrier()

    pltpu.sync_copy(deg.at[pl.ds(base, CHUNK)], disv)

    @pl.loop(0, CHUNK // L)
    def _(i):
        sl = pl.ds(i * L, L)
        disv[sl] = _rsqrt16(disv[sl])

    @pl.loop(0, NRB)
    def _(i):
        row0 = base + i * RB
        pltpu.sync_copy(emb2.at[c].at[pl.ds(row0, RB)], abuf)
        pltpu.sync_copy(abuf, acc_out.at[c].at[pl.ds(row0, RB)])

        @pl.loop(0, RB // L)
        def _(g):
            dvec = disv[pl.ds(i * RB + g * L, L)]
            for j in range(L):
                d = dvec[j]
                r = g * L + j
                for t in range(HALF // L):
                    sl = pl.ds(t * L, L)
                    abuf[r, sl] = d * abuf[r, sl]

        pltpu.sync_copy(abuf, y_hbm.at[c].at[pl.ds(row0, RB)])

    plsc.subcore_barrier()

    for layer in range(N_LAYERS):
        last = layer == N_LAYERS - 1

        with jax.named_scope(f"edge_{layer}"):
            @pl.loop(0, NSB)
            def _(q):
                pltpu.sync_copy(srcp.at[s].at[q], sidx)
                pltpu.sync_copy(dstp.at[s].at[q], didx)
                gd = [None] * SB
                sd = [None] * SB
                for b in range(2):
                    gd[b] = pltpu.async_copy(
                        y_hbm.at[c].at[sidx.at[b]], gbufs[b], gsems[b])
                for b in range(SB):
                    r = b % NGB
                    gd[b].wait()
                    if b >= 2:
                        sd[b - 2].wait()
                    sd[b] = pltpu.async_copy(
                        gbufs[r], accum.at[didx.at[b]], ssems[r], add=True)
                    if b + 2 < SB:
                        r2 = (b + 2) % NGB
                        gd[b + 2] = pltpu.async_copy(
                            y_hbm.at[c].at[sidx.at[b + 2]], gbufs[r2],
                            gsems[r2])
                sd[SB - 2].wait()
                sd[SB - 1].wait()

        plsc.subcore_barrier()

        if not last:
            @pl.loop(0, RB)
            def _(r):
                zv2 = jnp.zeros((L,), jnp.float32)
                for t in range(HALF // L):
                    g1[r, pl.ds(t * L, L)] = zv2

        with jax.named_scope(f"flush_{layer}"):
            @pl.loop(0, NRB)
            def _(i, _last=last):
                row0 = base + i * RB
                pltpu.sync_copy(accum.at[pl.ds(row0, RB)], abuf)
                if not _last:
                    pltpu.sync_copy(g1.at[pl.ds(0, RB)],
                                    accum.at[pl.ds(row0, RB)])
                pltpu.sync_copy(acc_out.at[c].at[pl.ds(row0, RB)], obuf)

                @pl.loop(0, RB // L)
                def _(g):
                    dvec = disv[pl.ds(i * RB + g * L, L)]
                    for j in range(L):
                        d = dvec[j]
                        r = g * L + j
                        for t in range(HALF // L):
                            sl = pl.ds(t * L, L)
                            x = d * abuf[r, sl]
                            obuf[r, sl] = obuf[r, sl] + x
                            if not _last:
                                abuf[r, sl] = d * x

                pltpu.sync_copy(obuf, acc_out.at[c].at[pl.ds(row0, RB)])
                if not _last:
                    pltpu.sync_copy(abuf, y_hbm.at[c].at[pl.ds(row0, RB)])

        if not last:
            plsc.subcore_barrier()


_prop = pl.kernel(
    _prop_body,
    out_type=(
        jax.ShapeDtypeStruct((NC, NP, HALF), jnp.float32),
        jax.ShapeDtypeStruct((NC, NP, HALF), jnp.float32),
    ),
    mesh=plsc.VectorSubcoreMesh(core_axis_name="c", subcore_axis_name="s",
                                num_cores=NC, num_subcores=NS),
    scratch_types=[
        pltpu.VMEM_SHARED((NP, HALF), jnp.float32),
        pltpu.VMEM_SHARED((NP,), jnp.float32),
        pltpu.VMEM((SB, K), jnp.int32),
        pltpu.VMEM((SB, K), jnp.int32),
        pltpu.VMEM((K, HALF), jnp.float32),
        pltpu.VMEM((K, HALF), jnp.float32),
        pltpu.VMEM((K, HALF), jnp.float32),
        pltpu.VMEM((K, HALF), jnp.float32),
        pltpu.VMEM((K,), jnp.float32),
        pltpu.VMEM((CHUNK,), jnp.float32),
        pltpu.VMEM((RB, HALF), jnp.float32),
        pltpu.VMEM((RB, HALF), jnp.float32),
        pltpu.SemaphoreType.DMA,
        pltpu.SemaphoreType.DMA,
        pltpu.SemaphoreType.DMA,
        pltpu.SemaphoreType.DMA,
        pltpu.SemaphoreType.DMA,
        pltpu.SemaphoreType.DMA,
        pltpu.SemaphoreType.DMA,
        pltpu.SemaphoreType.DMA,
    ],
    compiler_params=pltpu.CompilerParams(needs_layout_passes=False),
    name="lightgcn_prop",
)


def _score_body(uix, mix, acc2, out, uv, mv, gul, gml, guh, gmh, sres):
    c = lax.axis_index("c")
    s = lax.axis_index("s")
    wid = s * NC + c

    pltpu.sync_copy(uix.at[wid], uv)
    pltpu.sync_copy(mix.at[wid], mv)
    for b in range(PB):
        pltpu.sync_copy(acc2.at[0].at[uv.at[b]], gul)
        pltpu.sync_copy(acc2.at[0].at[mv.at[b]], gml)
        pltpu.sync_copy(acc2.at[1].at[uv.at[b]], guh)
        pltpu.sync_copy(acc2.at[1].at[mv.at[b]], gmh)

        @pl.loop(0, K)
        def _(p):
            sl = pl.ds(0, L)
            t0 = gul[p, sl] * gml[p, sl] + guh[p, sl] * gmh[p, sl]
            for t in range(1, HALF // L):
                sl = pl.ds(t * L, L)
                t0 = t0 + gul[p, sl] * gml[p, sl] + guh[p, sl] * gmh[p, sl]
            val = jnp.sum(t0) * (1.0 / ((N_LAYERS + 1) * (N_LAYERS + 1)))
            idx = jnp.full((L,), p, jnp.int32)
            msk = lax.iota(jnp.int32, L) == lax.rem(p, L)
            plsc.store_scatter(sres, [idx], jnp.full((L,), val), mask=msk)

        pltpu.sync_copy(sres, out.at[pl.ds((wid * PB + b) * K, K)])


_score = pl.kernel(
    _score_body,
    out_type=jax.ShapeDtypeStruct((N_BATCH,), jnp.float32),
    mesh=plsc.VectorSubcoreMesh(core_axis_name="c", subcore_axis_name="s",
                                num_cores=NC, num_subcores=NS),
    scratch_types=[
        pltpu.VMEM((PB, K), jnp.int32),
        pltpu.VMEM((PB, K), jnp.int32),
        pltpu.VMEM((K, HALF), jnp.float32),
        pltpu.VMEM((K, HALF), jnp.float32),
        pltpu.VMEM((K, HALF), jnp.float32),
        pltpu.VMEM((K, HALF), jnp.float32),
        pltpu.VMEM((K,), jnp.float32),
    ],
    compiler_params=pltpu.CompilerParams(needs_layout_passes=False),
    name="lightgcn_score",
)


@jax.jit
def kernel(user_indices, movie_indices, edge_index, emb_weight):
    src = edge_index[0]
    dst = edge_index[1]
    pad = jnp.broadcast_to(
        N_NODES + (jnp.arange(NPADE, dtype=jnp.int32) % (NP - N_NODES)),
        (NS, NPADE))
    srcp = jnp.concatenate([src.reshape(NS, EREAL), pad],
                           axis=1).reshape(NS, NSB, SB, K)
    dstp = jnp.concatenate([dst.reshape(NS, EREAL), pad],
                           axis=1).reshape(NS, NSB, SB, K)

    emb2 = jnp.zeros((NC, NP, HALF), jnp.float32)
    emb2 = emb2.at[0, :N_NODES].set(emb_weight[:, :HALF])
    emb2 = emb2.at[1, :N_NODES].set(emb_weight[:, HALF:])

    acc, _ = _prop(srcp, dstp, emb2)

    uix = user_indices.reshape(NC * NS, PB, K)
    mix = movie_indices.reshape(NC * NS, PB, K)
    return _score(uix, mix, acc)


# speedup vs baseline: 1.0261x; 1.0261x over previous
import jax
import jax.numpy as jnp
from jax import lax
from jax.experimental import pallas as pl
from jax.experimental.pallas import tpu as pltpu
from jax.experimental.pallas import tpu_sc as plsc

N_NODES = 10000
HALF = 128
N_EDGES = 160000
N_BATCH = 16384
N_LAYERS = 3

NC = 2
NS = 16
L = 16

NP = 10240
CHUNK = NP // NS
RB = 32
NRB = CHUNK // RB
K = 128
SB = 8
EPW = 10240
NB = EPW // K
NSB = NB // SB
EPAD = EPW * NS
EREAL = N_EDGES // NS
NPADE = EPW - EREAL

PPW = N_BATCH // (NC * NS)
PB = PPW // K


def _rsqrt16(v):
    iv = plsc.bitcast(v, jnp.int32)
    iv = jnp.int32(0x5F3759DF) - (iv >> 1)
    y = plsc.bitcast(iv, jnp.float32)
    for _ in range(3):
        y = y * (1.5 - 0.5 * v * y * y)
    return jnp.where(v > 0.0, y, 0.0)


def _prop_body(srcp, dstp, emb2, acc_out, y_hbm,
               accum, deg, sidx, didx, g0, g1, onesb, disv,
               abuf, obuf, sg0, sg1, ss0, ss1, sra, sro, swa, swy, sz0, sz1):
    c = lax.axis_index("c")
    s = lax.axis_index("s")
    base = s * CHUNK

    one = jnp.ones((L,), jnp.float32)
    zv = jnp.zeros((L,), jnp.float32)
    for t in range(K // L):
        onesb[pl.ds(t * L, L)] = one

    @pl.loop(0, K)
    def _(r):
        for t in range(HALF // L):
            g0[r, pl.ds(t * L, L)] = zv

    @pl.loop(0, (CHUNK + L) // L)
    def _(i):
        disv[pl.ds(i * L, L)] = zv

    pltpu.sync_copy(disv.at[pl.ds(0, CHUNK)], deg.at[pl.ds(base, CHUNK)])

    for i in range(CHUNK // K):
        pltpu.sync_copy(g0, accum.at[pl.ds(base + i * K, K)])

    plsc.subcore_barrier()

    with jax.named_scope("deg_phase"):
        @pl.loop(0, NSB)
        def _(q):
            pltpu.sync_copy(dstp.at[s].at[q], didx)
            for b in range(SB):
                pltpu.sync_copy(onesb, deg.at[didx.at[b]], add=True)

    plsc.subcore_barrier()

    pltpu.sync_copy(deg.at[pl.ds(base, CHUNK)], disv.at[pl.ds(0, CHUNK)])

    @pl.loop(0, CHUNK // L)
    def _(i):
        sl = pl.ds(i * L, L)
        disv[sl] = _rsqrt16(disv[sl])

    @pl.loop(0, NRB)
    def _(i):
        row0 = base + i * RB
        pltpu.sync_copy(emb2.at[c].at[pl.ds(row0, RB)], abuf)
        pltpu.sync_copy(abuf, acc_out.at[c].at[pl.ds(row0, RB)])

        @pl.loop(0, RB // L)
        def _(g):
            dvec = disv[pl.ds(i * RB + g * L, L)]
            for j in range(L):
                d = dvec[j]
                r = g * L + j
                for t in range(HALF // L):
                    sl = pl.ds(t * L, L)
                    abuf[r, sl] = d * abuf[r, sl]

        pltpu.sync_copy(abuf, y_hbm.at[c].at[pl.ds(row0, RB)])

    plsc.subcore_barrier()

    for layer in range(N_LAYERS):
        last = layer == N_LAYERS - 1

        with jax.named_scope(f"edge_{layer}"):
            @pl.loop(0, NSB)
            def _(q):
                pltpu.sync_copy(srcp.at[s].at[q], sidx)
                pltpu.sync_copy(dstp.at[s].at[q], didx)
                bufs = ((g0, sg0, ss0), (g1, sg1, ss1))
                gd = [pltpu.async_copy(y_hbm.at[c].at[sidx.at[0]], g0, sg0),
                      None]
                sd = [None, None]
                for b in range(SB):
                    gb, _, gsc = bufs[b % 2]
                    nb, ngs, _ = bufs[(b + 1) % 2]
                    gd[b % 2].wait()
                    if b > 0:
                        sd[(b - 1) % 2].wait()
                    if b + 1 < SB:
                        gd[(b + 1) % 2] = pltpu.async_copy(
                            y_hbm.at[c].at[sidx.at[b + 1]], nb, ngs)
                    sd[b % 2] = pltpu.async_copy(
                        gb, accum.at[didx.at[b]], gsc, add=True)
                sd[(SB - 1) % 2].wait()

        plsc.subcore_barrier()

        if not last:
            @pl.loop(0, RB)
            def _(r):
                zv2 = jnp.zeros((L,), jnp.float32)
                for t in range(HALF // L):
                    g1[r, pl.ds(t * L, L)] = zv2

        with jax.named_scope(f"flush_{layer}"):
            wprev = []
            zprev = [None, None]
            for i in range(NRB):
                row0 = base + i * RB
                for dsc in wprev:
                    dsc.wait()
                da = pltpu.async_copy(accum.at[pl.ds(row0, RB)], abuf, sra)
                do = pltpu.async_copy(acc_out.at[c].at[pl.ds(row0, RB)],
                                      obuf, sro)
                da.wait()
                do.wait()
                if not last:
                    if zprev[i % 2] is not None:
                        zprev[i % 2].wait()
                    zprev[i % 2] = pltpu.async_copy(
                        g1.at[pl.ds(0, RB)], accum.at[pl.ds(row0, RB)],
                        (sz0, sz1)[i % 2])

                @pl.loop(0, RB)
                def _(r, _i=i, _last=last):
                    dv = disv[pl.ds(_i * RB + r, L)]
                    d = dv[0]
                    for t in range(HALF // L):
                        sl = pl.ds(t * L, L)
                        x = d * abuf[r, sl]
                        obuf[r, sl] = obuf[r, sl] + x
                        if not _last:
                            abuf[r, sl] = d * x

                wprev = [pltpu.async_copy(
                    obuf, acc_out.at[c].at[pl.ds(row0, RB)], swa)]
                if not last:
                    wprev.append(pltpu.async_copy(
                        abuf, y_hbm.at[c].at[pl.ds(row0, RB)], swy))
            for dsc in wprev:
                dsc.wait()
            for zp in zprev:
                if zp is not None:
                    zp.wait()

        if not last:
            plsc.subcore_barrier()


_prop = pl.kernel(
    _prop_body,
    out_type=(
        jax.ShapeDtypeStruct((NC, NP, HALF), jnp.float32),
        jax.ShapeDtypeStruct((NC, NP, HALF), jnp.float32),
    ),
    mesh=plsc.VectorSubcoreMesh(core_axis_name="c", subcore_axis_name="s",
                                num_cores=NC, num_subcores=NS),
    scratch_types=[
        pltpu.VMEM_SHARED((NP, HALF), jnp.float32),
        pltpu.VMEM_SHARED((NP,), jnp.float32),
        pltpu.VMEM((SB, K), jnp.int32),
        pltpu.VMEM((SB, K), jnp.int32),
        pltpu.VMEM((K, HALF), jnp.float32),
        pltpu.VMEM((K, HALF), jnp.float32),
        pltpu.VMEM((K,), jnp.float32),
        pltpu.VMEM((CHUNK + L,), jnp.float32),
        pltpu.VMEM((RB, HALF), jnp.float32),
        pltpu.VMEM((RB, HALF), jnp.float32),
        pltpu.SemaphoreType.DMA,
        pltpu.SemaphoreType.DMA,
        pltpu.SemaphoreType.DMA,
        pltpu.SemaphoreType.DMA,
        pltpu.SemaphoreType.DMA,
        pltpu.SemaphoreType.DMA,
        pltpu.SemaphoreType.DMA,
        pltpu.SemaphoreType.DMA,
        pltpu.SemaphoreType.DMA,
        pltpu.SemaphoreType.DMA,
    ],
    compiler_params=pltpu.CompilerParams(needs_layout_passes=False),
    name="lightgcn_prop",
)


def _score_body(uix, mix, acc2, out, uv, mv, gul, gml, guh, gmh, sres):
    c = lax.axis_index("c")
    s = lax.axis_index("s")
    wid = s * NC + c

    pltpu.sync_copy(uix.at[wid], uv)
    pltpu.sync_copy(mix.at[wid], mv)
    for b in range(PB):
        pltpu.sync_copy(acc2.at[0].at[uv.at[b]], gul)
        pltpu.sync_copy(acc2.at[0].at[mv.at[b]], gml)
        pltpu.sync_copy(acc2.at[1].at[uv.at[b]], guh)
        pltpu.sync_copy(acc2.at[1].at[mv.at[b]], gmh)

        @pl.loop(0, K)
        def _(p):
            sl = pl.ds(0, L)
            t0 = gul[p, sl] * gml[p, sl] + guh[p, sl] * gmh[p, sl]
            for t in range(1, HALF // L):
                sl = pl.ds(t * L, L)
                t0 = t0 + gul[p, sl] * gml[p, sl] + guh[p, sl] * gmh[p, sl]
            val = jnp.sum(t0) * (1.0 / ((N_LAYERS + 1) * (N_LAYERS + 1)))
            idx = jnp.full((L,), p, jnp.int32)
            msk = lax.iota(jnp.int32, L) == lax.rem(p, L)
            plsc.store_scatter(sres, [idx], jnp.full((L,), val), mask=msk)

        pltpu.sync_copy(sres, out.at[pl.ds((wid * PB + b) * K, K)])


_score = pl.kernel(
    _score_body,
    out_type=jax.ShapeDtypeStruct((N_BATCH,), jnp.float32),
    mesh=plsc.VectorSubcoreMesh(core_axis_name="c", subcore_axis_name="s",
                                num_cores=NC, num_subcores=NS),
    scratch_types=[
        pltpu.VMEM((PB, K), jnp.int32),
        pltpu.VMEM((PB, K), jnp.int32),
        pltpu.VMEM((K, HALF), jnp.float32),
        pltpu.VMEM((K, HALF), jnp.float32),
        pltpu.VMEM((K, HALF), jnp.float32),
        pltpu.VMEM((K, HALF), jnp.float32),
        pltpu.VMEM((K,), jnp.float32),
    ],
    compiler_params=pltpu.CompilerParams(needs_layout_passes=False),
    name="lightgcn_score",
)


@jax.jit
def kernel(user_indices, movie_indices, edge_index, emb_weight):
    src = edge_index[0]
    dst = edge_index[1]
    pad = jnp.broadcast_to(
        N_NODES + (jnp.arange(NPADE, dtype=jnp.int32) % (NP - N_NODES)),
        (NS, NPADE))
    srcp = jnp.concatenate([src.reshape(NS, EREAL), pad],
                           axis=1).reshape(NS, NSB, SB, K)
    dstp = jnp.concatenate([dst.reshape(NS, EREAL), pad],
                           axis=1).reshape(NS, NSB, SB, K)

    emb2 = jnp.zeros((NC, NP, HALF), jnp.float32)
    emb2 = emb2.at[0, :N_NODES].set(emb_weight[:, :HALF])
    emb2 = emb2.at[1, :N_NODES].set(emb_weight[:, HALF:])

    acc, _ = _prop(srcp, dstp, emb2)

    uix = user_indices.reshape(NC * NS, PB, K)
    mix = movie_indices.reshape(NC * NS, PB, K)
    return _score(uix, mix, acc)


# speedup vs baseline: 1.0545x; 1.0276x over previous
import jax
import jax.numpy as jnp
from jax import lax
from jax.experimental import pallas as pl
from jax.experimental.pallas import tpu as pltpu
from jax.experimental.pallas import tpu_sc as plsc

N_NODES = 10000
HALF = 128
N_EDGES = 160000
N_BATCH = 16384
N_LAYERS = 3

NC = 2
NS = 16
L = 16

NP = 10240
CHUNK = NP // NS
RB = 32
NRB = CHUNK // RB
K = 128
SB = 8
EPW = 10240
NB = EPW // K
NSB = NB // SB
EPAD = EPW * NS
EREAL = N_EDGES // NS
NPADE = EPW - EREAL

PPW = N_BATCH // (NC * NS)
P2 = 64
PB = PPW // P2


def _rsqrt16(v):
    iv = plsc.bitcast(v, jnp.int32)
    iv = jnp.int32(0x5F3759DF) - (iv >> 1)
    y = plsc.bitcast(iv, jnp.float32)
    for _ in range(3):
        y = y * (1.5 - 0.5 * v * y * y)
    return jnp.where(v > 0.0, y, 0.0)


def _prop_body(srcp, dstp, emb2, acc_out, y_hbm,
               accum, deg, sidx, didx, g0, g1, onesb, disv,
               abuf, obuf, sg0, sg1, ss0, ss1, sra, sro, swa, swy, sz0, sz1):
    c = lax.axis_index("c")
    s = lax.axis_index("s")
    base = s * CHUNK

    one = jnp.ones((L,), jnp.float32)
    zv = jnp.zeros((L,), jnp.float32)
    for t in range(K // L):
        onesb[pl.ds(t * L, L)] = one

    @pl.loop(0, K)
    def _(r):
        for t in range(HALF // L):
            g0[r, pl.ds(t * L, L)] = zv

    @pl.loop(0, (CHUNK + L) // L)
    def _(i):
        disv[pl.ds(i * L, L)] = zv

    pltpu.sync_copy(disv.at[pl.ds(0, CHUNK)], deg.at[pl.ds(base, CHUNK)])

    for i in range(CHUNK // K):
        pltpu.sync_copy(g0, accum.at[pl.ds(base + i * K, K)])

    plsc.subcore_barrier()

    with jax.named_scope("deg_phase"):
        @pl.loop(0, NSB)
        def _(q):
            pltpu.sync_copy(dstp.at[s].at[q], didx)
            for b in range(SB):
                pltpu.sync_copy(onesb, deg.at[didx.at[b]], add=True)

    plsc.subcore_barrier()

    pltpu.sync_copy(deg.at[pl.ds(base, CHUNK)], disv.at[pl.ds(0, CHUNK)])

    @pl.loop(0, CHUNK // L)
    def _(i):
        sl = pl.ds(i * L, L)
        disv[sl] = _rsqrt16(disv[sl])

    @pl.loop(0, NRB)
    def _(i):
        row0 = base + i * RB
        pltpu.sync_copy(emb2.at[c].at[pl.ds(row0, RB)], abuf)
        pltpu.sync_copy(abuf, acc_out.at[c].at[pl.ds(row0, RB)])

        @pl.loop(0, RB // L)
        def _(g):
            dvec = disv[pl.ds(i * RB + g * L, L)]
            for j in range(L):
                d = dvec[j]
                r = g * L + j
                for t in range(HALF // L):
                    sl = pl.ds(t * L, L)
                    abuf[r, sl] = d * abuf[r, sl]

        pltpu.sync_copy(abuf, y_hbm.at[c].at[pl.ds(row0, RB)])

    plsc.subcore_barrier()

    for layer in range(N_LAYERS):
        last = layer == N_LAYERS - 1

        with jax.named_scope(f"edge_{layer}"):
            @pl.loop(0, NSB)
            def _(q):
                pltpu.sync_copy(srcp.at[s].at[q], sidx)
                pltpu.sync_copy(dstp.at[s].at[q], didx)
                bufs = ((g0, sg0, ss0), (g1, sg1, ss1))
                gd = [pltpu.async_copy(y_hbm.at[c].at[sidx.at[0]], g0, sg0),
                      None]
                sd = [None, None]
                for b in range(SB):
                    gb, _, gsc = bufs[b % 2]
                    nb, ngs, _ = bufs[(b + 1) % 2]
                    gd[b % 2].wait()
                    if b > 0:
                        sd[(b - 1) % 2].wait()
                    if b + 1 < SB:
                        gd[(b + 1) % 2] = pltpu.async_copy(
                            y_hbm.at[c].at[sidx.at[b + 1]], nb, ngs)
                    sd[b % 2] = pltpu.async_copy(
                        gb, accum.at[didx.at[b]], gsc, add=True)
                sd[(SB - 1) % 2].wait()

        plsc.subcore_barrier()

        if not last:
            @pl.loop(0, RB)
            def _(r):
                zv2 = jnp.zeros((L,), jnp.float32)
                for t in range(HALF // L):
                    g1[r, pl.ds(t * L, L)] = zv2

        with jax.named_scope(f"flush_{layer}"):
            wprev = []
            zprev = [None, None]
            for i in range(NRB):
                row0 = base + i * RB
                for dsc in wprev:
                    dsc.wait()
                da = pltpu.async_copy(accum.at[pl.ds(row0, RB)], abuf, sra)
                do = pltpu.async_copy(acc_out.at[c].at[pl.ds(row0, RB)],
                                      obuf, sro)
                da.wait()
                do.wait()
                if not last:
                    if zprev[i % 2] is not None:
                        zprev[i % 2].wait()
                    zprev[i % 2] = pltpu.async_copy(
                        g1.at[pl.ds(0, RB)], accum.at[pl.ds(row0, RB)],
                        (sz0, sz1)[i % 2])

                @pl.loop(0, RB)
                def _(r, _i=i, _last=last):
                    dv = disv[pl.ds(_i * RB + r, L)]
                    d = dv[0]
                    for t in range(HALF // L):
                        sl = pl.ds(t * L, L)
                        x = d * abuf[r, sl]
                        obuf[r, sl] = obuf[r, sl] + x
                        if not _last:
                            abuf[r, sl] = d * x

                wprev = [pltpu.async_copy(
                    obuf, acc_out.at[c].at[pl.ds(row0, RB)], swa)]
                if not last:
                    wprev.append(pltpu.async_copy(
                        abuf, y_hbm.at[c].at[pl.ds(row0, RB)], swy))
            for dsc in wprev:
                dsc.wait()
            for zp in zprev:
                if zp is not None:
                    zp.wait()

        if not last:
            plsc.subcore_barrier()


_prop = pl.kernel(
    _prop_body,
    out_type=(
        jax.ShapeDtypeStruct((NC, NP, HALF), jnp.float32),
        jax.ShapeDtypeStruct((NC, NP, HALF), jnp.float32),
    ),
    mesh=plsc.VectorSubcoreMesh(core_axis_name="c", subcore_axis_name="s",
                                num_cores=NC, num_subcores=NS),
    scratch_types=[
        pltpu.VMEM_SHARED((NP, HALF), jnp.float32),
        pltpu.VMEM_SHARED((NP,), jnp.float32),
        pltpu.VMEM((SB, K), jnp.int32),
        pltpu.VMEM((SB, K), jnp.int32),
        pltpu.VMEM((K, HALF), jnp.float32),
        pltpu.VMEM((K, HALF), jnp.float32),
        pltpu.VMEM((K,), jnp.float32),
        pltpu.VMEM((CHUNK + L,), jnp.float32),
        pltpu.VMEM((RB, HALF), jnp.float32),
        pltpu.VMEM((RB, HALF), jnp.float32),
        pltpu.SemaphoreType.DMA,
        pltpu.SemaphoreType.DMA,
        pltpu.SemaphoreType.DMA,
        pltpu.SemaphoreType.DMA,
        pltpu.SemaphoreType.DMA,
        pltpu.SemaphoreType.DMA,
        pltpu.SemaphoreType.DMA,
        pltpu.SemaphoreType.DMA,
        pltpu.SemaphoreType.DMA,
        pltpu.SemaphoreType.DMA,
    ],
    compiler_params=pltpu.CompilerParams(needs_layout_passes=False),
    name="lightgcn_prop",
)


def _score_body(uix, mix, acc2, out, uv, mv,
                gul0, gml0, guh0, gmh0, gul1, gml1, guh1, gmh1, sres,
                sga, sgb):
    c = lax.axis_index("c")
    s = lax.axis_index("s")
    wid = s * NC + c

    pltpu.sync_copy(uix.at[wid], uv)
    pltpu.sync_copy(mix.at[wid], mv)

    bufs = ((gul0, gml0, guh0, gmh0, sga), (gul1, gml1, guh1, gmh1, sgb))

    def fire(b):
        ul, ml, uh, mh, sem = bufs[b % 2]
        return [pltpu.async_copy(acc2.at[0].at[uv.at[b]], ul, sem),
                pltpu.async_copy(acc2.at[0].at[mv.at[b]], ml, sem),
                pltpu.async_copy(acc2.at[1].at[uv.at[b]], uh, sem),
                pltpu.async_copy(acc2.at[1].at[mv.at[b]], mh, sem)]

    pend = {0: fire(0)}
    for b in range(PB):
        if b + 1 < PB:
            pend[b + 1] = fire(b + 1)
        for dsc in pend.pop(b):
            dsc.wait()
        ul, ml, uh, mh, _ = bufs[b % 2]

        @pl.loop(0, P2)
        def _(p, ul=ul, ml=ml, uh=uh, mh=mh):
            sl = pl.ds(0, L)
            t0 = ul[p, sl] * ml[p, sl] + uh[p, sl] * mh[p, sl]
            for t in range(1, HALF // L):
                sl = pl.ds(t * L, L)
                t0 = t0 + ul[p, sl] * ml[p, sl] + uh[p, sl] * mh[p, sl]
            val = jnp.sum(t0) * (1.0 / ((N_LAYERS + 1) * (N_LAYERS + 1)))
            idx = jnp.full((L,), p, jnp.int32)
            msk = lax.iota(jnp.int32, L) == lax.rem(p, L)
            plsc.store_scatter(sres, [idx], jnp.full((L,), val), mask=msk)

        pltpu.sync_copy(sres, out.at[pl.ds(wid * PPW + b * P2, P2)])


_score = pl.kernel(
    _score_body,
    out_type=jax.ShapeDtypeStruct((N_BATCH,), jnp.float32),
    mesh=plsc.VectorSubcoreMesh(core_axis_name="c", subcore_axis_name="s",
                                num_cores=NC, num_subcores=NS),
    scratch_types=[
        pltpu.VMEM((PB, P2), jnp.int32),
        pltpu.VMEM((PB, P2), jnp.int32),
        pltpu.VMEM((P2, HALF), jnp.float32),
        pltpu.VMEM((P2, HALF), jnp.float32),
        pltpu.VMEM((P2, HALF), jnp.float32),
        pltpu.VMEM((P2, HALF), jnp.float32),
        pltpu.VMEM((P2, HALF), jnp.float32),
        pltpu.VMEM((P2, HALF), jnp.float32),
        pltpu.VMEM((P2, HALF), jnp.float32),
        pltpu.VMEM((P2, HALF), jnp.float32),
        pltpu.VMEM((P2,), jnp.float32),
        pltpu.SemaphoreType.DMA,
        pltpu.SemaphoreType.DMA,
    ],
    compiler_params=pltpu.CompilerParams(needs_layout_passes=False),
    name="lightgcn_score",
)


@jax.jit
def kernel(user_indices, movie_indices, edge_index, emb_weight):
    src = edge_index[0]
    dst = edge_index[1]
    pad = jnp.broadcast_to(
        N_NODES + (jnp.arange(NPADE, dtype=jnp.int32) % (NP - N_NODES)),
        (NS, NPADE))
    srcp = jnp.concatenate([src.reshape(NS, EREAL), pad],
                           axis=1).reshape(NS, NSB, SB, K)
    dstp = jnp.concatenate([dst.reshape(NS, EREAL), pad],
                           axis=1).reshape(NS, NSB, SB, K)

    emb2 = jnp.zeros((NC, NP, HALF), jnp.float32)
    emb2 = emb2.at[0, :N_NODES].set(emb_weight[:, :HALF])
    emb2 = emb2.at[1, :N_NODES].set(emb_weight[:, HALF:])

    acc, _ = _prop(srcp, dstp, emb2)

    uix = user_indices.reshape(NC * NS, PB, P2)
    mix = movie_indices.reshape(NC * NS, PB, P2)
    return _score(uix, mix, acc)
